# baseline (speedup 1.0000x reference)
import functools

import jax
import jax.numpy as jnp
from jax import lax
from jax.experimental import pallas as pl
from jax.experimental.pallas import tpu as pltpu
from jax.experimental.pallas import tpu_sc as plsc

B = 16384
F = 26
V = 100000
D = 16
NI = 13

CB = 64
CBF = CB * F
NSTREAM = CBF // 128


def _fm_sc(cate_flat, int_flat, emb1f, emb2f, wpad):
    info = plsc.get_sparse_core_info()
    NC, NS = info.num_cores, info.num_subcores
    NW = NC * NS
    spw = B // NW
    nchunks = spw // CB
    mesh = plsc.VectorSubcoreMesh(core_axis_name="c", subcore_axis_name="s")

    @functools.partial(
        pl.kernel,
        mesh=mesh,
        out_type=jax.ShapeDtypeStruct((B * D,), jnp.float32),
        compiler_params=pltpu.CompilerParams(use_tc_tiling_on_sc=False),
        scratch_types=[
            pltpu.VMEM((CBF,), jnp.int32),
            pltpu.VMEM((CBF, D), jnp.float32),
            pltpu.VMEM((CBF,), jnp.float32),
            pltpu.VMEM((CB * D,), jnp.float32),
            pltpu.VMEM((CB * D,), jnp.float32),
            pltpu.VMEM((16,), jnp.float32),
            pltpu.SemaphoreType.DMA,
        ],
    )
    def k(cate_hbm, int_hbm, e1_hbm, e2_hbm, w_hbm, out_hbm,
          idx_v, rows_v, e1_v, int_v, out_v, w_v, sem):
        wid = lax.axis_index("s") * NC + lax.axis_index("c")
        iota = lax.iota(jnp.int32, 16)
        e1mask = iota >= (2 * D - F)

        pltpu.sync_copy(w_hbm, w_v)
        wv = w_v[...]

        def chunk_body(c, _):
            sbase = wid * spw + c * CB

            pltpu.sync_copy(cate_hbm.at[pl.ds(sbase * F, CBF)], idx_v)

            def off_t(t, _):
                f = lax.rem(t * 16 + iota, F)
                sl = pl.ds(t * 16, 16)
                idx_v[sl] = idx_v[sl] + f * V
                return ()
            lax.fori_loop(0, CBF // 16, off_t, ())

            cps = []
            for i in range(NSTREAM):
                cps.append(pltpu.async_copy(
                    e2_hbm.at[idx_v.at[pl.ds(i * 128, 128)]],
                    rows_v.at[pl.ds(i * 128, 128)], sem))
            for i in range(NSTREAM):
                cps.append(pltpu.async_copy(
                    e1_hbm.at[idx_v.at[pl.ds(i * 128, 128)]],
                    e1_v.at[pl.ds(i * 128, 128)], sem))
            pltpu.sync_copy(int_hbm.at[pl.ds(sbase * D, CB * D)], int_v)
            for cp in cps:
                cp.wait()

            def sample_body(s, _):
                rr = s * F
                e = rows_v[rr]
                acc = e
                ssq = e * e
                for f in range(1, F):
                    e = rows_v[rr + f]
                    acc = acc + e
                    ssq = ssq + e * e
                e1a = e1_v[pl.ds(rr, D)]
                e1b = e1_v[pl.ds(rr + F - D, D)]
                tot = (0.5 * (acc * acc - ssq)
                       + int_v[pl.ds(s * D, D)] * wv
                       + e1a + jnp.where(e1mask, e1b, 0.0))
                out_v[pl.ds(s * D, D)] = tot
                return ()
            lax.fori_loop(0, CB, sample_body, ())

            pltpu.sync_copy(out_v, out_hbm.at[pl.ds(sbase * D, CB * D)])
            return ()

        lax.fori_loop(0, nchunks, chunk_body, ())

    return k(cate_flat, int_flat, emb1f, emb2f, wpad)


def _finish_tc(partials):
    x2d = partials.reshape(B * D // 128, 128)

    def body(x_ref, o_ref):
        x = x_ref[...]
        kk = lax.broadcasted_iota(jnp.int32, (128, 8), 0)
        mm = lax.broadcasted_iota(jnp.int32, (128, 8), 1)
        mat = (kk // D == mm).astype(jnp.float32)
        s = jnp.dot(x, mat, preferred_element_type=jnp.float32)
        o_ref[...] = 1.0 / (1.0 + jnp.exp(-s))

    return pl.pallas_call(
        body,
        out_shape=jax.ShapeDtypeStruct((B * D // 128, 8), jnp.float32),
    )(x2d)


def kernel(cate_data, int_data, emb1, emb2, W1, b1):
    cate_flat = cate_data.reshape(B * F)
    emb1f = emb1.reshape(F * V)
    emb2f = emb2.reshape(F * V, D)
    int_pad = jnp.concatenate(
        [int_data,
         jnp.zeros((B, D - NI - 1), jnp.float32),
         jnp.ones((B, 1), jnp.float32)], axis=1).reshape(B * D)
    wpad = jnp.concatenate(
        [W1[:, 0], jnp.zeros((D - NI - 1,), jnp.float32), b1])
    partials = _fm_sc(cate_flat, int_pad, emb1f, emb2f, wpad)
    y = _finish_tc(partials)
    return y.reshape(B, 1)


# speedup vs baseline: 1.0957x; 1.0957x over previous
import functools

import jax
import jax.numpy as jnp
from jax import lax
from jax.experimental import pallas as pl
from jax.experimental.pallas import tpu as pltpu
from jax.experimental.pallas import tpu_sc as plsc

B = 16384
F = 26
V = 100000
D = 16
NI = 13

CB = 16
CBF = CB * F
NSTREAM = 4
SLEN = CBF // NSTREAM

NVC = (V + 127) // 128
TW = 512


def _repack_tc(e2v):
    def body(x_ref, o_ref):
        x = x_ref[...]
        xp = jnp.concatenate(
            [x, jnp.zeros((TW - F * D, 128), jnp.float32)], axis=0)
        o_ref[...] = jnp.transpose(xp)

    return pl.pallas_call(
        body,
        grid=(NVC,),
        in_specs=[pl.BlockSpec((F * D, 128), lambda v: (0, v))],
        out_specs=pl.BlockSpec((128, TW), lambda v: (v, 0)),
        out_shape=jax.ShapeDtypeStruct((V, TW), jnp.float32),
    )(e2v)


def _fm_sc(cate_flat, int_flat, emb1f, emb2p, wpad):
    info = plsc.get_sparse_core_info()
    NC, NS = info.num_cores, info.num_subcores
    NW = NC * NS
    spw = B // NW
    nchunks = spw // CB
    mesh = plsc.VectorSubcoreMesh(core_axis_name="c", subcore_axis_name="s")

    @functools.partial(
        pl.kernel,
        mesh=mesh,
        out_type=jax.ShapeDtypeStruct((B * D,), jnp.float32),
        scratch_types=[
            pltpu.VMEM((CBF,), jnp.int32),
            pltpu.VMEM((CBF,), jnp.int32),
            pltpu.VMEM((CBF, 128), jnp.float32),
            pltpu.VMEM((CBF,), jnp.float32),
            pltpu.VMEM((CB * D,), jnp.float32),
            pltpu.VMEM((CB * D,), jnp.float32),
            pltpu.VMEM((16,), jnp.float32),
            pltpu.SemaphoreType.DMA,
        ],
    )
    def k(cate_hbm, int_hbm, e1_hbm, e2_hbm, w_hbm, out_hbm,
          idx_v, idx1_v, rows_v, e1_v, int_v, out_v, w_v, sem):
        wid = lax.axis_index("s") * NC + lax.axis_index("c")
        iota = lax.iota(jnp.int32, 16)
        e1mask = iota >= (2 * D - F)

        pltpu.sync_copy(w_hbm, w_v)
        wv = w_v[...]

        def chunk_body(c, _):
            sbase = wid * spw + c * CB

            pltpu.sync_copy(cate_hbm.at[pl.ds(sbase * F, CBF)], idx_v)

            def off_t(t, _):
                f = lax.rem(t * 16 + iota, F)
                sl = pl.ds(t * 16, 16)
                u = idx_v[sl]
                idx1_v[sl] = u + f * V
                idx_v[sl] = u * (TW // 128) + (f >> 3)
                return ()
            lax.fori_loop(0, CBF // 16, off_t, ())

            cps = []
            for i in range(NSTREAM):
                cps.append(pltpu.async_copy(
                    e2_hbm.at[idx_v.at[pl.ds(i * SLEN, SLEN)]],
                    rows_v.at[pl.ds(i * SLEN, SLEN)], sem))
            for i in range(NSTREAM):
                cps.append(pltpu.async_copy(
                    e1_hbm.at[idx1_v.at[pl.ds(i * SLEN, SLEN)]],
                    e1_v.at[pl.ds(i * SLEN, SLEN)], sem))
            pltpu.sync_copy(int_hbm.at[pl.ds(sbase * D, CB * D)], int_v)
            for cp in cps:
                cp.wait()

            def sample_body(s, _):
                rr = s * F
                acc = rows_v[rr, pl.ds(0, D)]
                ssq = acc * acc
                for f in range(1, F):
                    e = rows_v[rr + f, pl.ds((f % 8) * D, D)]
                    acc = acc + e
                    ssq = ssq + e * e
                e1a = e1_v[pl.ds(rr, D)]
                e1b = e1_v[pl.ds(rr + F - D, D)]
                tot = (0.5 * (acc * acc - ssq)
                       + int_v[pl.ds(s * D, D)] * wv
                       + e1a + jnp.where(e1mask, e1b, 0.0))
                out_v[pl.ds(s * D, D)] = tot
                return ()
            lax.fori_loop(0, CB, sample_body, ())

            pltpu.sync_copy(out_v, out_hbm.at[pl.ds(sbase * D, CB * D)])
            return ()

        lax.fori_loop(0, nchunks, chunk_body, ())

    return k(cate_flat, int_flat, emb1f, emb2p, wpad)


def _finish_tc(partials):
    x2d = partials.reshape(B * D // 128, 128)

    def body(x_ref, o_ref):
        x = x_ref[...]
        kk = lax.broadcasted_iota(jnp.int32, (128, 8), 0)
        mm = lax.broadcasted_iota(jnp.int32, (128, 8), 1)
        mat = (kk // D == mm).astype(jnp.float32)
        s = jnp.dot(x, mat, preferred_element_type=jnp.float32)
        o_ref[...] = 1.0 / (1.0 + jnp.exp(-s))

    return pl.pallas_call(
        body,
        out_shape=jax.ShapeDtypeStruct((B * D // 128, 8), jnp.float32),
    )(x2d)


def kernel(cate_data, int_data, emb1, emb2, W1, b1):
    cate_flat = cate_data.reshape(B * F)
    emb1f = emb1.reshape(F * V)
    e2v = emb2.transpose(0, 2, 1).reshape(F * D, V)
    emb2p = _repack_tc(e2v).reshape(V * TW // 128, 128)
    int_pad = jnp.concatenate(
        [int_data,
         jnp.zeros((B, D - NI - 1), jnp.float32),
         jnp.ones((B, 1), jnp.float32)], axis=1).reshape(B * D)
    wpad = jnp.concatenate(
        [W1[:, 0], jnp.zeros((D - NI - 1,), jnp.float32), b1])
    partials = _fm_sc(cate_flat, int_pad, emb1f, emb2p, wpad)
    y = _finish_tc(partials)
    return y.reshape(B, 1)


# speedup vs baseline: 1.5473x; 1.4122x over previous
import functools

import jax
import jax.numpy as jnp
from jax import lax
from jax.experimental import pallas as pl
from jax.experimental.pallas import tpu as pltpu
from jax.experimental.pallas import tpu_sc as plsc

B = 16384
F = 26
V = 100000
D = 16
NI = 13

CB = 16
CBF = CB * F
NSTREAM = 4
SLEN = CBF // NSTREAM

NVC = (V + 127) // 128
TW = 512


def _repack_tc(e2v):
    VC = 512

    def body(x_ref, o_ref):
        x = x_ref[...]
        xp = jnp.concatenate(
            [x, jnp.zeros((TW - F * D, VC), jnp.float32)], axis=0)
        o_ref[...] = jnp.transpose(xp)

    return pl.pallas_call(
        body,
        grid=((V + VC - 1) // VC,),
        in_specs=[pl.BlockSpec((F * D, VC), lambda v: (0, v))],
        out_specs=pl.BlockSpec((VC, TW), lambda v: (v, 0)),
        out_shape=jax.ShapeDtypeStruct((V, TW), jnp.float32),
    )(e2v)


def _fm_sc(cate_flat, int_flat, emb1f, emb2p, wpad):
    info = plsc.get_sparse_core_info()
    NC, NS = info.num_cores, info.num_subcores
    NW = NC * NS
    spw = B // NW
    nchunks = spw // CB
    mesh = plsc.VectorSubcoreMesh(core_axis_name="c", subcore_axis_name="s")

    @functools.partial(
        pl.kernel,
        mesh=mesh,
        out_type=jax.ShapeDtypeStruct((B * D,), jnp.float32),
        scratch_types=[
            pltpu.VMEM((CBF,), jnp.int32),
            pltpu.VMEM((CBF,), jnp.int32),
            pltpu.VMEM((CBF, 128), jnp.float32),
            pltpu.VMEM((CBF,), jnp.float32),
            pltpu.VMEM((CB * D,), jnp.float32),
            pltpu.VMEM((CB * D,), jnp.float32),
            pltpu.VMEM((16,), jnp.float32),
            pltpu.SemaphoreType.DMA,
        ],
    )
    def k(cate_hbm, int_hbm, e1_hbm, e2_hbm, w_hbm, out_hbm,
          idx_v, idx1_v, rows_v, e1_v, int_v, out_v, w_v, sem):
        wid = lax.axis_index("s") * NC + lax.axis_index("c")
        iota = lax.iota(jnp.int32, 16)
        e1mask = iota >= (2 * D - F)

        pltpu.sync_copy(w_hbm, w_v)
        wv = w_v[...]

        def chunk_body(c, _):
            sbase = wid * spw + c * CB

            pltpu.sync_copy(cate_hbm.at[pl.ds(sbase * F, CBF)], idx_v)

            def off_t(t, _):
                f = lax.rem(t * 16 + iota, F)
                sl = pl.ds(t * 16, 16)
                u = idx_v[sl]
                idx1_v[sl] = u + f * V
                idx_v[sl] = u * (TW // 128) + (f >> 3)
                return ()
            lax.fori_loop(0, CBF // 16, off_t, ())

            cps = []
            for i in range(NSTREAM):
                cps.append(pltpu.async_copy(
                    e2_hbm.at[idx_v.at[pl.ds(i * SLEN, SLEN)]],
                    rows_v.at[pl.ds(i * SLEN, SLEN)], sem))
            for i in range(NSTREAM):
                cps.append(pltpu.async_copy(
                    e1_hbm.at[idx1_v.at[pl.ds(i * SLEN, SLEN)]],
                    e1_v.at[pl.ds(i * SLEN, SLEN)], sem))
            pltpu.sync_copy(int_hbm.at[pl.ds(sbase * D, CB * D)], int_v)
            for cp in cps:
                cp.wait()

            def sample_body(s, _):
                rr = s * F
                acc = rows_v[rr, pl.ds(0, D)]
                ssq = acc * acc
                for f in range(1, F):
                    e = rows_v[rr + f, pl.ds((f % 8) * D, D)]
                    acc = acc + e
                    ssq = ssq + e * e
                e1a = e1_v[pl.ds(rr, D)]
                e1b = e1_v[pl.ds(rr + F - D, D)]
                tot = (0.5 * (acc * acc - ssq)
                       + int_v[pl.ds(s * D, D)] * wv
                       + e1a + jnp.where(e1mask, e1b, 0.0))
                out_v[pl.ds(s * D, D)] = tot
                return ()
            lax.fori_loop(0, CB, sample_body, ())

            pltpu.sync_copy(out_v, out_hbm.at[pl.ds(sbase * D, CB * D)])
            return ()

        lax.fori_loop(0, nchunks, chunk_body, ())

    return k(cate_flat, int_flat, emb1f, emb2p, wpad)


def _finish_tc(partials):
    x2d = partials.reshape(B * D // 128, 128)

    def body(x_ref, o_ref):
        x = x_ref[...]
        kk = lax.broadcasted_iota(jnp.int32, (128, 8), 0)
        mm = lax.broadcasted_iota(jnp.int32, (128, 8), 1)
        mat = (kk // D == mm).astype(jnp.float32)
        s = jnp.dot(x, mat, preferred_element_type=jnp.float32)
        o_ref[...] = 1.0 / (1.0 + jnp.exp(-s))

    return pl.pallas_call(
        body,
        out_shape=jax.ShapeDtypeStruct((B * D // 128, 8), jnp.float32),
    )(x2d)


def kernel(cate_data, int_data, emb1, emb2, W1, b1):
    cate_flat = cate_data.reshape(B * F)
    emb1f = emb1.reshape(F * V)
    e2v = emb2.transpose(0, 2, 1).reshape(F * D, V)
    emb2p = _repack_tc(e2v).reshape(V * TW // 128, 128)
    int_pad = jnp.concatenate(
        [int_data,
         jnp.zeros((B, D - NI - 1), jnp.float32),
         jnp.ones((B, 1), jnp.float32)], axis=1).reshape(B * D)
    wpad = jnp.concatenate(
        [W1[:, 0], jnp.zeros((D - NI - 1,), jnp.float32), b1])
    partials = _fm_sc(cate_flat, int_pad, emb1f, emb2p, wpad)
    y = _finish_tc(partials)
    return y.reshape(B, 1)


# speedup vs baseline: 1.7365x; 1.1223x over previous
import functools

import jax
import jax.numpy as jnp
from jax import lax
from jax.experimental import pallas as pl
from jax.experimental.pallas import tpu as pltpu
from jax.experimental.pallas import tpu_sc as plsc

B = 16384
F = 26
V = 100000
D = 16
NI = 13

CB = 16
CBF = CB * F
NSTREAM = 4
SLEN = CBF // NSTREAM

NVC = (V + 127) // 128
TW = 512


def _repack_tc(e2v):
    VC = 2048

    def body(x_ref, o_ref):
        x = x_ref[...]
        xp = jnp.concatenate(
            [x, jnp.zeros((TW - F * D, VC), jnp.float32)], axis=0)
        o_ref[...] = jnp.transpose(xp)

    return pl.pallas_call(
        body,
        grid=((V + VC - 1) // VC,),
        in_specs=[pl.BlockSpec((F * D, VC), lambda v: (0, v))],
        out_specs=pl.BlockSpec((VC, TW), lambda v: (v, 0)),
        out_shape=jax.ShapeDtypeStruct((V, TW), jnp.float32),
    )(e2v)


def _fm_sc(cate_flat, int_flat, emb1f, emb2p, wpad):
    info = plsc.get_sparse_core_info()
    NC, NS = info.num_cores, info.num_subcores
    NW = NC * NS
    spw = B // NW
    nchunks = spw // CB
    mesh = plsc.VectorSubcoreMesh(core_axis_name="c", subcore_axis_name="s")

    @functools.partial(
        pl.kernel,
        mesh=mesh,
        out_type=jax.ShapeDtypeStruct((B * D,), jnp.float32),
        scratch_types=[
            pltpu.VMEM((CBF,), jnp.int32),
            pltpu.VMEM((CBF,), jnp.int32),
            pltpu.VMEM((CBF, 128), jnp.float32),
            pltpu.VMEM((CBF,), jnp.float32),
            pltpu.VMEM((CB * D,), jnp.float32),
            pltpu.VMEM((CB * D,), jnp.float32),
            pltpu.VMEM((16,), jnp.float32),
            pltpu.SemaphoreType.DMA,
        ],
    )
    def k(cate_hbm, int_hbm, e1_hbm, e2_hbm, w_hbm, out_hbm,
          idx_v, idx1_v, rows_v, e1_v, int_v, out_v, w_v, sem):
        wid = lax.axis_index("s") * NC + lax.axis_index("c")
        iota = lax.iota(jnp.int32, 16)
        e1mask = iota >= (2 * D - F)

        pltpu.sync_copy(w_hbm, w_v)
        wv = w_v[...]

        def chunk_body(c, _):
            sbase = wid * spw + c * CB

            pltpu.sync_copy(cate_hbm.at[pl.ds(sbase * F, CBF)], idx_v)

            def off_t(t, _):
                f = lax.rem(t * 16 + iota, F)
                sl = pl.ds(t * 16, 16)
                u = idx_v[sl]
                idx1_v[sl] = u + f * V
                idx_v[sl] = u * (TW // 128) + (f >> 3)
                return ()
            lax.fori_loop(0, CBF // 16, off_t, ())

            cps = []
            for i in range(NSTREAM):
                cps.append(pltpu.async_copy(
                    e2_hbm.at[idx_v.at[pl.ds(i * SLEN, SLEN)]],
                    rows_v.at[pl.ds(i * SLEN, SLEN)], sem))
            for i in range(NSTREAM):
                cps.append(pltpu.async_copy(
                    e1_hbm.at[idx1_v.at[pl.ds(i * SLEN, SLEN)]],
                    e1_v.at[pl.ds(i * SLEN, SLEN)], sem))
            pltpu.sync_copy(int_hbm.at[pl.ds(sbase * D, CB * D)], int_v)
            for cp in cps:
                cp.wait()

            def sample_body(s, _):
                rr = s * F
                acc = rows_v[rr, pl.ds(0, D)]
                ssq = acc * acc
                for f in range(1, F):
                    e = rows_v[rr + f, pl.ds((f % 8) * D, D)]
                    acc = acc + e
                    ssq = ssq + e * e
                e1a = e1_v[pl.ds(rr, D)]
                e1b = e1_v[pl.ds(rr + F - D, D)]
                tot = (0.5 * (acc * acc - ssq)
                       + int_v[pl.ds(s * D, D)] * wv
                       + e1a + jnp.where(e1mask, e1b, 0.0))
                out_v[pl.ds(s * D, D)] = tot
                return ()
            lax.fori_loop(0, CB, sample_body, ())

            pltpu.sync_copy(out_v, out_hbm.at[pl.ds(sbase * D, CB * D)])
            return ()

        lax.fori_loop(0, nchunks, chunk_body, ())

    return k(cate_flat, int_flat, emb1f, emb2p, wpad)


def _finish_tc(partials):
    x2d = partials.reshape(B * D // 128, 128)

    def body(x_ref, o_ref):
        x = x_ref[...]
        kk = lax.broadcasted_iota(jnp.int32, (128, 8), 0)
        mm = lax.broadcasted_iota(jnp.int32, (128, 8), 1)
        mat = (kk // D == mm).astype(jnp.float32)
        s = jnp.dot(x, mat, preferred_element_type=jnp.float32)
        o_ref[...] = 1.0 / (1.0 + jnp.exp(-s))

    return pl.pallas_call(
        body,
        out_shape=jax.ShapeDtypeStruct((B * D // 128, 8), jnp.float32),
    )(x2d)


def kernel(cate_data, int_data, emb1, emb2, W1, b1):
    cate_flat = cate_data.reshape(B * F)
    emb1f = emb1.reshape(F * V)
    e2v = emb2.transpose(0, 2, 1).reshape(F * D, V)
    emb2p = _repack_tc(e2v).reshape(V * TW // 128, 128)
    int_pad = jnp.concatenate(
        [int_data,
         jnp.zeros((B, D - NI - 1), jnp.float32),
         jnp.ones((B, 1), jnp.float32)], axis=1).reshape(B * D)
    wpad = jnp.concatenate(
        [W1[:, 0], jnp.zeros((D - NI - 1,), jnp.float32), b1])
    partials = _fm_sc(cate_flat, int_pad, emb1f, emb2p, wpad)
    y = _finish_tc(partials)
    return y.reshape(B, 1)


# speedup vs baseline: 1.7557x; 1.0111x over previous
import functools

import jax
import jax.numpy as jnp
from jax import lax
from jax.experimental import pallas as pl
from jax.experimental.pallas import tpu as pltpu
from jax.experimental.pallas import tpu_sc as plsc

B = 16384
F = 26
V = 100000
D = 16
NI = 13

CB = 16
CBF = CB * F
NSTREAM = 4
SLEN = CBF // NSTREAM

NVC = (V + 127) // 128
TW = 512


def _repack_tc(e2v):
    VC = 4096

    def body(x_ref, o_ref):
        x = x_ref[...]
        o_ref[:, pl.ds(0, F * D)] = jnp.transpose(x)

    return pl.pallas_call(
        body,
        grid=((V + VC - 1) // VC,),
        in_specs=[pl.BlockSpec((F * D, VC), lambda v: (0, v))],
        out_specs=pl.BlockSpec((VC, TW), lambda v: (v, 0)),
        out_shape=jax.ShapeDtypeStruct((V, TW), jnp.float32),
    )(e2v)


def _fm_sc(cate_flat, int_flat, emb1f, emb2p, wpad):
    info = plsc.get_sparse_core_info()
    NC, NS = info.num_cores, info.num_subcores
    NW = NC * NS
    spw = B // NW
    nchunks = spw // CB
    mesh = plsc.VectorSubcoreMesh(core_axis_name="c", subcore_axis_name="s")

    @functools.partial(
        pl.kernel,
        mesh=mesh,
        out_type=jax.ShapeDtypeStruct((B * D,), jnp.float32),
        scratch_types=[
            pltpu.VMEM((CBF,), jnp.int32),
            pltpu.VMEM((CBF,), jnp.int32),
            pltpu.VMEM((CBF, 128), jnp.float32),
            pltpu.VMEM((CBF,), jnp.float32),
            pltpu.VMEM((CB * D,), jnp.float32),
            pltpu.VMEM((CB * D,), jnp.float32),
            pltpu.VMEM((16,), jnp.float32),
            pltpu.SemaphoreType.DMA,
        ],
    )
    def k(cate_hbm, int_hbm, e1_hbm, e2_hbm, w_hbm, out_hbm,
          idx_v, idx1_v, rows_v, e1_v, int_v, out_v, w_v, sem):
        wid = lax.axis_index("s") * NC + lax.axis_index("c")
        iota = lax.iota(jnp.int32, 16)
        e1mask = iota >= (2 * D - F)

        pltpu.sync_copy(w_hbm, w_v)
        wv = w_v[...]

        def chunk_body(c, _):
            sbase = wid * spw + c * CB

            pltpu.sync_copy(cate_hbm.at[pl.ds(sbase * F, CBF)], idx_v)

            def off_t(t, _):
                f = lax.rem(t * 16 + iota, F)
                sl = pl.ds(t * 16, 16)
                u = idx_v[sl]
                idx1_v[sl] = u + f * V
                idx_v[sl] = u * (TW // 128) + (f >> 3)
                return ()
            lax.fori_loop(0, CBF // 16, off_t, ())

            cps = []
            for i in range(NSTREAM):
                cps.append(pltpu.async_copy(
                    e2_hbm.at[idx_v.at[pl.ds(i * SLEN, SLEN)]],
                    rows_v.at[pl.ds(i * SLEN, SLEN)], sem))
            for i in range(NSTREAM):
                cps.append(pltpu.async_copy(
                    e1_hbm.at[idx1_v.at[pl.ds(i * SLEN, SLEN)]],
                    e1_v.at[pl.ds(i * SLEN, SLEN)], sem))
            pltpu.sync_copy(int_hbm.at[pl.ds(sbase * D, CB * D)], int_v)
            for cp in cps:
                cp.wait()

            def sample_body(s, _):
                rr = s * F
                acc = rows_v[rr, pl.ds(0, D)]
                ssq = acc * acc
                for f in range(1, F):
                    e = rows_v[rr + f, pl.ds((f % 8) * D, D)]
                    acc = acc + e
                    ssq = ssq + e * e
                e1a = e1_v[pl.ds(rr, D)]
                e1b = e1_v[pl.ds(rr + F - D, D)]
                tot = (0.5 * (acc * acc - ssq)
                       + int_v[pl.ds(s * D, D)] * wv
                       + e1a + jnp.where(e1mask, e1b, 0.0))
                out_v[pl.ds(s * D, D)] = tot
                return ()
            lax.fori_loop(0, CB, sample_body, ())

            pltpu.sync_copy(out_v, out_hbm.at[pl.ds(sbase * D, CB * D)])
            return ()

        lax.fori_loop(0, nchunks, chunk_body, ())

    return k(cate_flat, int_flat, emb1f, emb2p, wpad)


def _finish_tc(partials):
    x2d = partials.reshape(B * D // 128, 128)

    def body(x_ref, o_ref):
        x = x_ref[...]
        kk = lax.broadcasted_iota(jnp.int32, (128, 8), 0)
        mm = lax.broadcasted_iota(jnp.int32, (128, 8), 1)
        mat = (kk // D == mm).astype(jnp.float32)
        s = jnp.dot(x, mat, preferred_element_type=jnp.float32)
        o_ref[...] = 1.0 / (1.0 + jnp.exp(-s))

    return pl.pallas_call(
        body,
        out_shape=jax.ShapeDtypeStruct((B * D // 128, 8), jnp.float32),
    )(x2d)


def kernel(cate_data, int_data, emb1, emb2, W1, b1):
    cate_flat = cate_data.reshape(B * F)
    emb1f = emb1.reshape(F * V)
    e2v = emb2.transpose(0, 2, 1).reshape(F * D, V)
    emb2p = _repack_tc(e2v).reshape(V * TW // 128, 128)
    int_pad = jnp.concatenate(
        [int_data,
         jnp.zeros((B, D - NI - 1), jnp.float32),
         jnp.ones((B, 1), jnp.float32)], axis=1).reshape(B * D)
    wpad = jnp.concatenate(
        [W1[:, 0], jnp.zeros((D - NI - 1,), jnp.float32), b1])
    partials = _fm_sc(cate_flat, int_pad, emb1f, emb2p, wpad)
    y = _finish_tc(partials)
    return y.reshape(B, 1)


# speedup vs baseline: 3.2478x; 1.8499x over previous
import functools

import jax
import jax.numpy as jnp
from jax import lax
from jax.experimental import pallas as pl
from jax.experimental.pallas import tpu as pltpu
from jax.experimental.pallas import tpu_sc as plsc

B = 16384
F = 26
V = 100000
D = 16
NI = 13

CB = 16
CBF = CB * F
NSTREAM = 4
SLEN = CBF // NSTREAM

NVC = (V + 127) // 128
TW = 512


def _repack_tc(e2v, e1v):
    VC = 2048

    def body(x_ref, e1_ref, o_ref, o1_ref):
        for j in range(3):
            o_ref[j] = jnp.transpose(x_ref[pl.ds(j * 128, 128), :])
        o_ref[3, :, pl.ds(0, F * D - 384)] = jnp.transpose(
            x_ref[pl.ds(384, F * D - 384), :])
        o1_ref[:, pl.ds(0, F)] = jnp.transpose(e1_ref[...])

    return pl.pallas_call(
        body,
        grid=((V + VC - 1) // VC,),
        in_specs=[pl.BlockSpec((F * D, VC), lambda v: (0, v)),
                  pl.BlockSpec((F, VC), lambda v: (0, v))],
        out_specs=[pl.BlockSpec((4, VC, 128), lambda v: (0, v, 0)),
                   pl.BlockSpec((VC, 128), lambda v: (v, 0))],
        out_shape=[jax.ShapeDtypeStruct((4, V, 128), jnp.float32),
                   jax.ShapeDtypeStruct((V, 128), jnp.float32)],
    )(e2v, e1v)


def _fm_sc(cate_flat, int_flat, emb1f, emb2p, wpad):
    info = plsc.get_sparse_core_info()
    NC, NS = info.num_cores, info.num_subcores
    NW = NC * NS
    spw = B // NW
    nchunks = spw // CB
    mesh = plsc.VectorSubcoreMesh(core_axis_name="c", subcore_axis_name="s")

    @functools.partial(
        pl.kernel,
        mesh=mesh,
        out_type=jax.ShapeDtypeStruct((B * D,), jnp.float32),
        scratch_types=[
            pltpu.VMEM((CBF,), jnp.int32),
            pltpu.VMEM((CBF,), jnp.int32),
            pltpu.VMEM((CBF, 128), jnp.float32),
            pltpu.VMEM((CBF,), jnp.float32),
            pltpu.VMEM((CB * D,), jnp.float32),
            pltpu.VMEM((CB * D,), jnp.float32),
            pltpu.VMEM((16,), jnp.float32),
            pltpu.SemaphoreType.DMA,
        ],
    )
    def k(cate_hbm, int_hbm, e1_hbm, e2_hbm, w_hbm, out_hbm,
          idx_v, idx1_v, rows_v, e1_v, int_v, out_v, w_v, sem):
        wid = lax.axis_index("s") * NC + lax.axis_index("c")
        iota = lax.iota(jnp.int32, 16)
        e1mask = iota >= (2 * D - F)

        pltpu.sync_copy(w_hbm, w_v)
        wv = w_v[...]

        def chunk_body(c, _):
            sbase = wid * spw + c * CB

            pltpu.sync_copy(cate_hbm.at[pl.ds(sbase * F, CBF)], idx_v)

            def off_t(t, _):
                f = lax.rem(t * 16 + iota, F)
                sl = pl.ds(t * 16, 16)
                u = idx_v[sl]
                idx1_v[sl] = u * 128 + f
                idx_v[sl] = (f >> 3) * V + u
                return ()
            lax.fori_loop(0, CBF // 16, off_t, ())

            cps = []
            for i in range(NSTREAM):
                cps.append(pltpu.async_copy(
                    e2_hbm.at[idx_v.at[pl.ds(i * SLEN, SLEN)]],
                    rows_v.at[pl.ds(i * SLEN, SLEN)], sem))
            for i in range(NSTREAM):
                cps.append(pltpu.async_copy(
                    e1_hbm.at[idx1_v.at[pl.ds(i * SLEN, SLEN)]],
                    e1_v.at[pl.ds(i * SLEN, SLEN)], sem))
            pltpu.sync_copy(int_hbm.at[pl.ds(sbase * D, CB * D)], int_v)
            for cp in cps:
                cp.wait()

            def sample_body(s, _):
                rr = s * F
                acc = rows_v[rr, pl.ds(0, D)]
                ssq = acc * acc
                for f in range(1, F):
                    e = rows_v[rr + f, pl.ds((f % 8) * D, D)]
                    acc = acc + e
                    ssq = ssq + e * e
                e1a = e1_v[pl.ds(rr, D)]
                e1b = e1_v[pl.ds(rr + F - D, D)]
                tot = (0.5 * (acc * acc - ssq)
                       + int_v[pl.ds(s * D, D)] * wv
                       + e1a + jnp.where(e1mask, e1b, 0.0))
                out_v[pl.ds(s * D, D)] = tot
                return ()
            lax.fori_loop(0, CB, sample_body, ())

            pltpu.sync_copy(out_v, out_hbm.at[pl.ds(sbase * D, CB * D)])
            return ()

        lax.fori_loop(0, nchunks, chunk_body, ())

    return k(cate_flat, int_flat, emb1f, emb2p, wpad)


def _finish_tc(partials):
    x2d = partials.reshape(B * D // 128, 128)

    def body(x_ref, o_ref):
        x = x_ref[...]
        kk = lax.broadcasted_iota(jnp.int32, (128, 8), 0)
        mm = lax.broadcasted_iota(jnp.int32, (128, 8), 1)
        mat = (kk // D == mm).astype(jnp.float32)
        s = jnp.dot(x, mat, preferred_element_type=jnp.float32)
        o_ref[...] = 1.0 / (1.0 + jnp.exp(-s))

    return pl.pallas_call(
        body,
        out_shape=jax.ShapeDtypeStruct((B * D // 128, 8), jnp.float32),
    )(x2d)


def kernel(cate_data, int_data, emb1, emb2, W1, b1):
    cate_flat = cate_data.reshape(B * F)
    e2v = emb2.transpose(0, 2, 1).reshape(F * D, V)
    packed, e1packed = _repack_tc(e2v, emb1[:, :, 0])
    emb2p = packed.reshape(4 * V, 128)
    emb1f = e1packed.reshape(V * 128)
    int_pad = jnp.concatenate(
        [int_data,
         jnp.zeros((B, D - NI - 1), jnp.float32),
         jnp.ones((B, 1), jnp.float32)], axis=1).reshape(B * D)
    wpad = jnp.concatenate(
        [W1[:, 0], jnp.zeros((D - NI - 1,), jnp.float32), b1])
    partials = _fm_sc(cate_flat, int_pad, emb1f, emb2p, wpad)
    y = _finish_tc(partials)
    return y.reshape(B, 1)


# speedup vs baseline: 3.7185x; 1.1449x over previous
import functools

import jax
import jax.numpy as jnp
from jax import lax
from jax.experimental import pallas as pl
from jax.experimental.pallas import tpu as pltpu
from jax.experimental.pallas import tpu_sc as plsc

B = 16384
F = 26
V = 100000
D = 16
NI = 13

CB = 16
CBF = CB * F
NSTREAM = 4
SLEN = CBF // NSTREAM

NVC = (V + 127) // 128
TW = 512


def _repack_tc(e2v, e1v):
    VC = 2048

    def body(x_ref, e1_ref, o_ref, o1_ref):
        for j in range(3):
            o_ref[j] = jnp.transpose(x_ref[pl.ds(j * 128, 128), :])
        o_ref[3, :, pl.ds(0, F * D - 384)] = jnp.transpose(
            x_ref[pl.ds(384, F * D - 384), :])
        o1_ref[:, pl.ds(0, F)] = jnp.transpose(e1_ref[...])

    return pl.pallas_call(
        body,
        grid=((V + VC - 1) // VC,),
        in_specs=[pl.BlockSpec((F * D, VC), lambda v: (0, v)),
                  pl.BlockSpec((F, VC), lambda v: (0, v))],
        out_specs=[pl.BlockSpec((4, VC, 128), lambda v: (0, v, 0)),
                   pl.BlockSpec((VC, 128), lambda v: (v, 0))],
        out_shape=[jax.ShapeDtypeStruct((4, V, 128), jnp.float32),
                   jax.ShapeDtypeStruct((V, 128), jnp.float32)],
    )(e2v, e1v)


def _fm_sc(cate_flat, int_flat, emb1f, emb2p, wpad):
    info = plsc.get_sparse_core_info()
    NC, NS = info.num_cores, info.num_subcores
    NW = NC * NS
    spw = B // NW
    nchunks = spw // CB
    mesh = plsc.VectorSubcoreMesh(core_axis_name="c", subcore_axis_name="s")

    @functools.partial(
        pl.kernel,
        mesh=mesh,
        out_type=jax.ShapeDtypeStruct((B * D,), jnp.float32),
        scratch_types=[
            pltpu.VMEM((2 * CBF,), jnp.int32),
            pltpu.VMEM((2 * CBF,), jnp.int32),
            pltpu.VMEM((2 * CBF, 128), jnp.float32),
            pltpu.VMEM((2 * CBF,), jnp.float32),
            pltpu.VMEM((2 * CB * D,), jnp.float32),
            pltpu.VMEM((CB * D,), jnp.float32),
            pltpu.VMEM((16,), jnp.float32),
            pltpu.SemaphoreType.DMA,
            pltpu.SemaphoreType.DMA,
        ],
    )
    def k(cate_hbm, int_hbm, e1_hbm, e2_hbm, w_hbm, out_hbm,
          idx_v, idx1_v, rows_v, e1_v, int_v, out_v, w_v, sem0, sem1):
        wid = lax.axis_index("s") * NC + lax.axis_index("c")
        iota = lax.iota(jnp.int32, 16)
        e1mask = iota >= (2 * D - F)
        sems = (sem0, sem1)

        pltpu.sync_copy(w_hbm, w_v)
        wv = w_v[...]

        def fire(p, c):
            sbase = wid * spw + c * CB
            pltpu.sync_copy(cate_hbm.at[pl.ds(sbase * F, CBF)],
                            idx_v.at[pl.ds(p * CBF, CBF)])

            def off_t(t, _):
                f = lax.rem(t * 16 + iota, F)
                sl = pl.ds(p * CBF + t * 16, 16)
                u = idx_v[sl]
                idx1_v[sl] = u * 128 + f
                idx_v[sl] = (f >> 3) * V + u
                return ()
            lax.fori_loop(0, CBF // 16, off_t, ())

            for i in range(NSTREAM):
                pltpu.async_copy(
                    e2_hbm.at[idx_v.at[pl.ds(p * CBF + i * SLEN, SLEN)]],
                    rows_v.at[pl.ds(p * CBF + i * SLEN, SLEN)], sems[p])
                pltpu.async_copy(
                    e1_hbm.at[idx1_v.at[pl.ds(p * CBF + i * SLEN, SLEN)]],
                    e1_v.at[pl.ds(p * CBF + i * SLEN, SLEN)], sems[p])
            pltpu.async_copy(int_hbm.at[pl.ds(sbase * D, CB * D)],
                             int_v.at[pl.ds(p * CB * D, CB * D)], sems[p])

        def drain(p):
            pltpu.make_async_copy(
                e2_hbm.at[pl.ds(0, CBF)],
                rows_v.at[pl.ds(p * CBF, CBF)], sems[p]).wait()
            pltpu.make_async_copy(
                e1_hbm.at[pl.ds(0, CBF)],
                e1_v.at[pl.ds(p * CBF, CBF)], sems[p]).wait()
            pltpu.make_async_copy(
                int_hbm.at[pl.ds(0, CB * D)],
                int_v.at[pl.ds(p * CB * D, CB * D)], sems[p]).wait()

        def compute(p, c):
            sbase = wid * spw + c * CB

            def sample_body(s, _):
                rr = p * CBF + s * F
                acc = rows_v[rr, pl.ds(0, D)]
                ssq = acc * acc
                for f in range(1, F):
                    e = rows_v[rr + f, pl.ds((f % 8) * D, D)]
                    acc = acc + e
                    ssq = ssq + e * e
                e1a = e1_v[pl.ds(rr, D)]
                e1b = e1_v[pl.ds(rr + F - D, D)]
                tot = (0.5 * (acc * acc - ssq)
                       + int_v[pl.ds(p * CB * D + s * D, D)] * wv
                       + e1a + jnp.where(e1mask, e1b, 0.0))
                out_v[pl.ds(s * D, D)] = tot
                return ()
            lax.fori_loop(0, CB, sample_body, ())

            pltpu.sync_copy(out_v, out_hbm.at[pl.ds(sbase * D, CB * D)])

        fire(0, 0)

        def pair_body(c2, _):
            fire(1, 2 * c2 + 1)
            drain(0)
            compute(0, 2 * c2)
            fire(0, lax.rem(2 * c2 + 2, nchunks))
            drain(1)
            compute(1, 2 * c2 + 1)
            return ()
        lax.fori_loop(0, nchunks // 2, pair_body, ())
        drain(0)

    return k(cate_flat, int_flat, emb1f, emb2p, wpad)


def _finish_tc(partials):
    x2d = partials.reshape(B * D // 128, 128)

    def body(x_ref, o_ref):
        x = x_ref[...]
        kk = lax.broadcasted_iota(jnp.int32, (128, 8), 0)
        mm = lax.broadcasted_iota(jnp.int32, (128, 8), 1)
        mat = (kk // D == mm).astype(jnp.float32)
        s = jnp.dot(x, mat, preferred_element_type=jnp.float32)
        o_ref[...] = 1.0 / (1.0 + jnp.exp(-s))

    return pl.pallas_call(
        body,
        out_shape=jax.ShapeDtypeStruct((B * D // 128, 8), jnp.float32),
    )(x2d)


def kernel(cate_data, int_data, emb1, emb2, W1, b1):
    cate_flat = cate_data.reshape(B * F)
    e2v = emb2.transpose(0, 2, 1).reshape(F * D, V)
    packed, e1packed = _repack_tc(e2v, emb1[:, :, 0])
    emb2p = packed.reshape(4 * V, 128)
    emb1f = e1packed.reshape(V * 128)
    int_pad = jnp.concatenate(
        [int_data,
         jnp.zeros((B, D - NI - 1), jnp.float32),
         jnp.ones((B, 1), jnp.float32)], axis=1).reshape(B * D)
    wpad = jnp.concatenate(
        [W1[:, 0], jnp.zeros((D - NI - 1,), jnp.float32), b1])
    partials = _fm_sc(cate_flat, int_pad, emb1f, emb2p, wpad)
    y = _finish_tc(partials)
    return y.reshape(B, 1)


# speedup vs baseline: 3.7865x; 1.0183x over previous
import functools

import jax
import jax.numpy as jnp
from jax import lax
from jax.experimental import pallas as pl
from jax.experimental.pallas import tpu as pltpu
from jax.experimental.pallas import tpu_sc as plsc

B = 16384
F = 26
V = 100000
D = 16
NI = 13

CB = 16
CBF = CB * F
NSTREAM = 4
SLEN = CBF // NSTREAM

NVC = (V + 127) // 128
TW = 512


def _repack_tc(e2v, e1v):
    VC = 4096

    def body(x_ref, e1_ref, o_ref, o1_ref):
        for j in range(3):
            o_ref[j] = jnp.transpose(x_ref[pl.ds(j * 128, 128), :])
        o_ref[3, :, pl.ds(0, F * D - 384)] = jnp.transpose(
            x_ref[pl.ds(384, F * D - 384), :])
        o1_ref[:, pl.ds(0, F)] = jnp.transpose(e1_ref[...])

    return pl.pallas_call(
        body,
        grid=((V + VC - 1) // VC,),
        in_specs=[pl.BlockSpec((F * D, VC), lambda v: (0, v)),
                  pl.BlockSpec((F, VC), lambda v: (0, v))],
        out_specs=[pl.BlockSpec((4, VC, 128), lambda v: (0, v, 0)),
                   pl.BlockSpec((VC, 128), lambda v: (v, 0))],
        out_shape=[jax.ShapeDtypeStruct((4, V, 128), jnp.float32),
                   jax.ShapeDtypeStruct((V, 128), jnp.float32)],
    )(e2v, e1v)


def _fm_sc(cate_flat, int_flat, emb1f, emb2p, wpad):
    info = plsc.get_sparse_core_info()
    NC, NS = info.num_cores, info.num_subcores
    NW = NC * NS
    spw = B // NW
    nchunks = spw // CB
    mesh = plsc.VectorSubcoreMesh(core_axis_name="c", subcore_axis_name="s")

    @functools.partial(
        pl.kernel,
        mesh=mesh,
        out_type=jax.ShapeDtypeStruct((B * D,), jnp.float32),
        scratch_types=[
            pltpu.VMEM((2 * CBF,), jnp.int32),
            pltpu.VMEM((2 * CBF,), jnp.int32),
            pltpu.VMEM((2 * CBF, 128), jnp.float32),
            pltpu.VMEM((2 * CBF,), jnp.float32),
            pltpu.VMEM((2 * CB * D,), jnp.float32),
            pltpu.VMEM((CB * D,), jnp.float32),
            pltpu.VMEM((16,), jnp.float32),
            pltpu.SemaphoreType.DMA,
            pltpu.SemaphoreType.DMA,
        ],
    )
    def k(cate_hbm, int_hbm, e1_hbm, e2_hbm, w_hbm, out_hbm,
          idx_v, idx1_v, rows_v, e1_v, int_v, out_v, w_v, sem0, sem1):
        wid = lax.axis_index("s") * NC + lax.axis_index("c")
        iota = lax.iota(jnp.int32, 16)
        e1mask = iota >= (2 * D - F)
        sems = (sem0, sem1)

        pltpu.sync_copy(w_hbm, w_v)
        wv = w_v[...]

        def fire(p, c):
            sbase = wid * spw + c * CB
            pltpu.sync_copy(cate_hbm.at[pl.ds(sbase * F, CBF)],
                            idx_v.at[pl.ds(p * CBF, CBF)])

            def off_t(t, _):
                f = lax.rem(t * 16 + iota, F)
                sl = pl.ds(p * CBF + t * 16, 16)
                u = idx_v[sl]
                idx1_v[sl] = u * 128 + f
                idx_v[sl] = (f >> 3) * V + u
                return ()
            lax.fori_loop(0, CBF // 16, off_t, ())

            for i in range(NSTREAM):
                pltpu.async_copy(
                    e2_hbm.at[idx_v.at[pl.ds(p * CBF + i * SLEN, SLEN)]],
                    rows_v.at[pl.ds(p * CBF + i * SLEN, SLEN)], sems[p])
                pltpu.async_copy(
                    e1_hbm.at[idx1_v.at[pl.ds(p * CBF + i * SLEN, SLEN)]],
                    e1_v.at[pl.ds(p * CBF + i * SLEN, SLEN)], sems[p])
            pltpu.async_copy(int_hbm.at[pl.ds(sbase * D, CB * D)],
                             int_v.at[pl.ds(p * CB * D, CB * D)], sems[p])

        def drain(p):
            pltpu.make_async_copy(
                e2_hbm.at[pl.ds(0, CBF)],
                rows_v.at[pl.ds(p * CBF, CBF)], sems[p]).wait()
            pltpu.make_async_copy(
                e1_hbm.at[pl.ds(0, CBF)],
                e1_v.at[pl.ds(p * CBF, CBF)], sems[p]).wait()
            pltpu.make_async_copy(
                int_hbm.at[pl.ds(0, CB * D)],
                int_v.at[pl.ds(p * CB * D, CB * D)], sems[p]).wait()

        def compute(p, c):
            sbase = wid * spw + c * CB

            def sample_body(s, _):
                rr = p * CBF + s * F
                acc = rows_v[rr, pl.ds(0, D)]
                ssq = acc * acc
                for f in range(1, F):
                    e = rows_v[rr + f, pl.ds((f % 8) * D, D)]
                    acc = acc + e
                    ssq = ssq + e * e
                e1a = e1_v[pl.ds(rr, D)]
                e1b = e1_v[pl.ds(rr + F - D, D)]
                tot = (0.5 * (acc * acc - ssq)
                       + int_v[pl.ds(p * CB * D + s * D, D)] * wv
                       + e1a + jnp.where(e1mask, e1b, 0.0))
                out_v[pl.ds(s * D, D)] = tot
                return ()
            lax.fori_loop(0, CB, sample_body, ())

            pltpu.sync_copy(out_v, out_hbm.at[pl.ds(sbase * D, CB * D)])

        fire(0, 0)

        def pair_body(c2, _):
            fire(1, 2 * c2 + 1)
            drain(0)
            compute(0, 2 * c2)
            fire(0, lax.rem(2 * c2 + 2, nchunks))
            drain(1)
            compute(1, 2 * c2 + 1)
            return ()
        lax.fori_loop(0, nchunks // 2, pair_body, ())
        drain(0)

    return k(cate_flat, int_flat, emb1f, emb2p, wpad)


def _finish_tc(partials):
    x2d = partials.reshape(B * D // 128, 128)

    def body(x_ref, o_ref):
        x = x_ref[...]
        kk = lax.broadcasted_iota(jnp.int32, (128, 8), 0)
        mm = lax.broadcasted_iota(jnp.int32, (128, 8), 1)
        mat = (kk // D == mm).astype(jnp.float32)
        s = jnp.dot(x, mat, preferred_element_type=jnp.float32)
        o_ref[...] = 1.0 / (1.0 + jnp.exp(-s))

    return pl.pallas_call(
        body,
        out_shape=jax.ShapeDtypeStruct((B * D // 128, 8), jnp.float32),
    )(x2d)


def kernel(cate_data, int_data, emb1, emb2, W1, b1):
    cate_flat = cate_data.reshape(B * F)
    e2v = emb2.transpose(0, 2, 1).reshape(F * D, V)
    packed, e1packed = _repack_tc(e2v, emb1[:, :, 0])
    emb2p = packed.reshape(4 * V, 128)
    emb1f = e1packed.reshape(V * 128)
    int_pad = jnp.concatenate(
        [int_data,
         jnp.zeros((B, D - NI - 1), jnp.float32),
         jnp.ones((B, 1), jnp.float32)], axis=1).reshape(B * D)
    wpad = jnp.concatenate(
        [W1[:, 0], jnp.zeros((D - NI - 1,), jnp.float32), b1])
    partials = _fm_sc(cate_flat, int_pad, emb1f, emb2p, wpad)
    y = _finish_tc(partials)
    return y.reshape(B, 1)


# speedup vs baseline: 3.7983x; 1.0031x over previous
import functools

import jax
import jax.numpy as jnp
from jax import lax
from jax.experimental import pallas as pl
from jax.experimental.pallas import tpu as pltpu
from jax.experimental.pallas import tpu_sc as plsc

B = 16384
F = 26
V = 100000
D = 16
NI = 13

CB = 16
CBF = CB * F
NSTREAM = 4
SLEN = CBF // NSTREAM

NVC = (V + 127) // 128
TW = 512


def _repack_tc(e2v, e1v):
    VC = 6144

    def body(x_ref, e1_ref, o_ref, o1_ref):
        for j in range(3):
            o_ref[j] = jnp.transpose(x_ref[pl.ds(j * 128, 128), :])
        o_ref[3, :, pl.ds(0, F * D - 384)] = jnp.transpose(
            x_ref[pl.ds(384, F * D - 384), :])
        o1_ref[:, pl.ds(0, F)] = jnp.transpose(e1_ref[...])

    return pl.pallas_call(
        body,
        grid=((V + VC - 1) // VC,),
        in_specs=[pl.BlockSpec((F * D, VC), lambda v: (0, v)),
                  pl.BlockSpec((F, VC), lambda v: (0, v))],
        out_specs=[pl.BlockSpec((4, VC, 128), lambda v: (0, v, 0)),
                   pl.BlockSpec((VC, 128), lambda v: (v, 0))],
        out_shape=[jax.ShapeDtypeStruct((4, V, 128), jnp.float32),
                   jax.ShapeDtypeStruct((V, 128), jnp.float32)],
    )(e2v, e1v)


def _fm_sc(cate_flat, int_flat, emb1f, emb2p, wpad):
    info = plsc.get_sparse_core_info()
    NC, NS = info.num_cores, info.num_subcores
    NW = NC * NS
    spw = B // NW
    nchunks = spw // CB
    mesh = plsc.VectorSubcoreMesh(core_axis_name="c", subcore_axis_name="s")

    @functools.partial(
        pl.kernel,
        mesh=mesh,
        out_type=jax.ShapeDtypeStruct((B * D,), jnp.float32),
        scratch_types=[
            pltpu.VMEM((2 * CBF,), jnp.int32),
            pltpu.VMEM((2 * CBF,), jnp.int32),
            pltpu.VMEM((2 * CBF, 128), jnp.float32),
            pltpu.VMEM((2 * CBF,), jnp.float32),
            pltpu.VMEM((2 * CB * D,), jnp.float32),
            pltpu.VMEM((CB * D,), jnp.float32),
            pltpu.VMEM((16,), jnp.float32),
            pltpu.SemaphoreType.DMA,
            pltpu.SemaphoreType.DMA,
        ],
    )
    def k(cate_hbm, int_hbm, e1_hbm, e2_hbm, w_hbm, out_hbm,
          idx_v, idx1_v, rows_v, e1_v, int_v, out_v, w_v, sem0, sem1):
        wid = lax.axis_index("s") * NC + lax.axis_index("c")
        iota = lax.iota(jnp.int32, 16)
        e1mask = iota >= (2 * D - F)
        sems = (sem0, sem1)

        pltpu.sync_copy(w_hbm, w_v)
        wv = w_v[...]

        def fire(p, c):
            sbase = wid * spw + c * CB
            pltpu.sync_copy(cate_hbm.at[pl.ds(sbase * F, CBF)],
                            idx_v.at[pl.ds(p * CBF, CBF)])

            def off_t(t, _):
                f = lax.rem(t * 16 + iota, F)
                sl = pl.ds(p * CBF + t * 16, 16)
                u = idx_v[sl]
                idx1_v[sl] = u * 128 + f
                idx_v[sl] = (f >> 3) * V + u
                return ()
            lax.fori_loop(0, CBF // 16, off_t, ())

            for i in range(NSTREAM):
                pltpu.async_copy(
                    e2_hbm.at[idx_v.at[pl.ds(p * CBF + i * SLEN, SLEN)]],
                    rows_v.at[pl.ds(p * CBF + i * SLEN, SLEN)], sems[p])
                pltpu.async_copy(
                    e1_hbm.at[idx1_v.at[pl.ds(p * CBF + i * SLEN, SLEN)]],
                    e1_v.at[pl.ds(p * CBF + i * SLEN, SLEN)], sems[p])
            pltpu.async_copy(int_hbm.at[pl.ds(sbase * D, CB * D)],
                             int_v.at[pl.ds(p * CB * D, CB * D)], sems[p])

        def drain(p):
            pltpu.make_async_copy(
                e2_hbm.at[pl.ds(0, CBF)],
                rows_v.at[pl.ds(p * CBF, CBF)], sems[p]).wait()
            pltpu.make_async_copy(
                e1_hbm.at[pl.ds(0, CBF)],
                e1_v.at[pl.ds(p * CBF, CBF)], sems[p]).wait()
            pltpu.make_async_copy(
                int_hbm.at[pl.ds(0, CB * D)],
                int_v.at[pl.ds(p * CB * D, CB * D)], sems[p]).wait()

        def compute(p, c):
            sbase = wid * spw + c * CB

            def sample_body(s, _):
                rr = p * CBF + s * F
                acc = rows_v[rr, pl.ds(0, D)]
                ssq = acc * acc
                for f in range(1, F):
                    e = rows_v[rr + f, pl.ds((f % 8) * D, D)]
                    acc = acc + e
                    ssq = ssq + e * e
                e1a = e1_v[pl.ds(rr, D)]
                e1b = e1_v[pl.ds(rr + F - D, D)]
                tot = (0.5 * (acc * acc - ssq)
                       + int_v[pl.ds(p * CB * D + s * D, D)] * wv
                       + e1a + jnp.where(e1mask, e1b, 0.0))
                out_v[pl.ds(s * D, D)] = tot
                return ()
            lax.fori_loop(0, CB, sample_body, ())

            pltpu.sync_copy(out_v, out_hbm.at[pl.ds(sbase * D, CB * D)])

        fire(0, 0)

        def pair_body(c2, _):
            fire(1, 2 * c2 + 1)
            drain(0)
            compute(0, 2 * c2)
            fire(0, lax.rem(2 * c2 + 2, nchunks))
            drain(1)
            compute(1, 2 * c2 + 1)
            return ()
        lax.fori_loop(0, nchunks // 2, pair_body, ())
        drain(0)

    return k(cate_flat, int_flat, emb1f, emb2p, wpad)


def _finish_tc(partials):
    x2d = partials.reshape(B * D // 128, 128)

    def body(x_ref, o_ref):
        x = x_ref[...]
        kk = lax.broadcasted_iota(jnp.int32, (128, 8), 0)
        mm = lax.broadcasted_iota(jnp.int32, (128, 8), 1)
        mat = (kk // D == mm).astype(jnp.float32)
        s = jnp.dot(x, mat, preferred_element_type=jnp.float32)
        o_ref[...] = 1.0 / (1.0 + jnp.exp(-s))

    return pl.pallas_call(
        body,
        out_shape=jax.ShapeDtypeStruct((B * D // 128, 8), jnp.float32),
    )(x2d)


def kernel(cate_data, int_data, emb1, emb2, W1, b1):
    cate_flat = cate_data.reshape(B * F)
    e2v = emb2.transpose(0, 2, 1).reshape(F * D, V)
    packed, e1packed = _repack_tc(e2v, emb1[:, :, 0])
    emb2p = packed.reshape(4 * V, 128)
    emb1f = e1packed.reshape(V * 128)
    int_pad = jnp.concatenate(
        [int_data,
         jnp.zeros((B, D - NI - 1), jnp.float32),
         jnp.ones((B, 1), jnp.float32)], axis=1).reshape(B * D)
    wpad = jnp.concatenate(
        [W1[:, 0], jnp.zeros((D - NI - 1,), jnp.float32), b1])
    partials = _fm_sc(cate_flat, int_pad, emb1f, emb2p, wpad)
    y = _finish_tc(partials)
    return y.reshape(B, 1)
